# initial kernel scaffold (unmeasured)
import jax
import jax.numpy as jnp
from jax import lax
from jax.experimental import pallas as pl
from jax.experimental.pallas import tpu as pltpu

N_DEV = 4
B, SQ, DM = 2, 512, 768
HQ, DH = 8, 64
SK = 512
NEG = -1e9
SCALE = 0.125


def kernel(x, Wq, K_ext, V_ext, Wo):
    def body(x_ref, wq_ref, k_ref, v_ref, wo_ref, out_ref,
             kv_ref, send_sems, recv_sems):
        my = lax.axis_index("i")
        left = (my + N_DEV - 1) % N_DEV
        right = (my + 1) % N_DEV

        barrier = pltpu.get_barrier_semaphore()
        for nbr in (left, right):
            pl.semaphore_signal(
                barrier, inc=1,
                device_id=(nbr,), device_id_type=pl.DeviceIdType.MESH,
            )
        pl.semaphore_wait(barrier, 2)

        kv_ref[3, 0] = k_ref[...].astype(jnp.bfloat16)
        kv_ref[3, 1] = v_ref[...].astype(jnp.bfloat16)

        def make_rdma(h):
            src = 3 if h == 0 else h - 1
            return pltpu.make_async_remote_copy(
                src_ref=kv_ref.at[src],
                dst_ref=kv_ref.at[h],
                send_sem=send_sems.at[h],
                recv_sem=recv_sems.at[h],
                device_id=(right,),
                device_id_type=pl.DeviceIdType.MESH,
            )

        rdmas = [make_rdma(0)]
        rdmas[0].start()

        wq = wq_ref[...].astype(jnp.bfloat16)
        qs = []
        for b in range(B):
            q = jnp.dot(x_ref[b].astype(jnp.bfloat16), wq,
                        preferred_element_type=jnp.float32)
            qs.append((q * SCALE).astype(jnp.bfloat16).reshape(SQ, HQ, DH))

        m = [[jnp.full((SQ, 1), -1e30, jnp.float32) for _ in range(HQ)]
             for _ in range(B)]
        l = [[jnp.zeros((SQ, 1), jnp.float32) for _ in range(HQ)]
             for _ in range(B)]
        acc = [[jnp.zeros((SQ, DH), jnp.float32) for _ in range(HQ)]
               for _ in range(B)]

        def accumulate(c, slot):
            qi = lax.broadcasted_iota(jnp.int32, (SQ, SK), 0)
            kj = lax.broadcasted_iota(jnp.int32, (SQ, SK), 1) + c * SK
            mask = (jnp.abs(qi - kj) <= 128) | (kj < 32) | (qi < 32)
            for b in range(B):
                for h in range(HQ):
                    k_bh = kv_ref[slot, 0, b, :, h, :]
                    v_bh = kv_ref[slot, 1, b, :, h, :]
                    s = lax.dot_general(
                        qs[b][:, h, :], k_bh, (((1,), (1,)), ((), ())),
                        preferred_element_type=jnp.float32)
                    s = jnp.where(mask, s, NEG)
                    m_new = jnp.maximum(m[b][h],
                                        jnp.max(s, axis=1, keepdims=True))
                    alpha = jnp.exp(m[b][h] - m_new)
                    p = jnp.exp(s - m_new)
                    l[b][h] = l[b][h] * alpha + jnp.sum(p, axis=1,
                                                        keepdims=True)
                    acc[b][h] = acc[b][h] * alpha + lax.dot_general(
                        p.astype(jnp.bfloat16), v_bh,
                        (((1,), (0,)), ((), ())),
                        preferred_element_type=jnp.float32)
                    m[b][h] = m_new

        accumulate(my, 3)

        for h in range(N_DEV - 1):
            rdmas[h].wait()
            if h < N_DEV - 2:
                nxt = make_rdma(h + 1)
                nxt.start()
                rdmas.append(nxt)
            accumulate((my + N_DEV - h - 1) % N_DEV, h)

        wo = wo_ref[...].astype(jnp.bfloat16)
        for b in range(B):
            ctx = jnp.concatenate(
                [(acc[b][h] / l[b][h]).astype(jnp.bfloat16)
                 for h in range(HQ)], axis=1)
            out_ref[b] = jnp.dot(ctx, wo, preferred_element_type=jnp.float32)

    return pl.pallas_call(
        body,
        out_shape=jax.ShapeDtypeStruct((B, SQ, DM), jnp.float32),
        in_specs=[pl.BlockSpec(memory_space=pltpu.VMEM)] * 5,
        out_specs=pl.BlockSpec(memory_space=pltpu.VMEM),
        scratch_shapes=[
            pltpu.VMEM((N_DEV, 2, B, SK, HQ, DH), jnp.bfloat16),
            pltpu.SemaphoreType.DMA((N_DEV - 1,)),
            pltpu.SemaphoreType.DMA((N_DEV - 1,)),
        ],
        compiler_params=pltpu.CompilerParams(collective_id=0),
    )(x, Wq, K_ext, V_ext, Wo)


# baseline (device time: 95157 ns/iter reference)
import jax
import jax.numpy as jnp
from jax import lax
from jax.experimental import pallas as pl
from jax.experimental.pallas import tpu as pltpu

N_DEV = 4
B, SQ, DM = 2, 512, 768
HQ, DH = 8, 64
HD = HQ * DH
SK = 512
NEG = -1e9
SCALE = 0.125
FIXED_MAX = 16.0


def kernel(x, Wq, K_ext, V_ext, Wo):
    Kb = K_ext.astype(jnp.bfloat16).reshape(B, SK, HD)
    Vb = V_ext.astype(jnp.bfloat16).reshape(B, SK, HD)

    def body(x_ref, wq_ref, k_ref, v_ref, wo_ref, out_ref,
             kv_ref, acc_ref, ksend, krecv, vsend, vrecv):
        my = lax.axis_index("i")
        left = (my + N_DEV - 1) % N_DEV
        right = (my + 1) % N_DEV

        barrier = pltpu.get_barrier_semaphore()
        for nbr in (left, right):
            pl.semaphore_signal(
                barrier, inc=1,
                device_id=(nbr,), device_id_type=pl.DeviceIdType.MESH,
            )
        pl.semaphore_wait(barrier, 2)

        rk0 = pltpu.make_async_remote_copy(
            src_ref=k_ref, dst_ref=kv_ref.at[0, 0],
            send_sem=ksend.at[0], recv_sem=krecv.at[0],
            device_id=(right,), device_id_type=pl.DeviceIdType.MESH)
        rv0 = pltpu.make_async_remote_copy(
            src_ref=v_ref, dst_ref=kv_ref.at[0, 1],
            send_sem=vsend.at[0], recv_sem=vrecv.at[0],
            device_id=(right,), device_id_type=pl.DeviceIdType.MESH)
        rk0.start()
        rv0.start()

        def make_fwd(h):
            return pltpu.make_async_remote_copy(
                src_ref=kv_ref.at[h - 1], dst_ref=kv_ref.at[h],
                send_sem=ksend.at[h], recv_sem=krecv.at[h],
                device_id=(right,), device_id_type=pl.DeviceIdType.MESH)

        wq = wq_ref[...].astype(jnp.bfloat16)
        qs = []
        for b in range(B):
            q = jnp.dot(x_ref[b].astype(jnp.bfloat16), wq,
                        preferred_element_type=jnp.float32)
            qs.append((q * SCALE).astype(jnp.bfloat16))

        acc_ref[...] = jnp.zeros((B, SQ, HQ * 128), jnp.float32)
        ones = jnp.ones((SK, 64), jnp.bfloat16)

        def accumulate(c, kat, vat):
            qi = lax.broadcasted_iota(jnp.int32, (SQ, SK), 0)
            kj = lax.broadcasted_iota(jnp.int32, (SQ, SK), 1) + c * SK
            mask = (jnp.abs(qi - kj) <= 128) | (kj < 32) | (qi < 32)
            for b in range(B):
                kb = kat(b)
                vb = vat(b)
                for h in range(HQ):
                    s = lax.dot_general(
                        qs[b][:, h * DH:(h + 1) * DH],
                        kb[:, h * DH:(h + 1) * DH],
                        (((1,), (1,)), ((), ())),
                        preferred_element_type=jnp.float32)
                    s = jnp.where(mask, s, NEG)
                    p = jnp.exp(s - FIXED_MAX).astype(jnp.bfloat16)
                    v_aug = jnp.concatenate(
                        [vb[:, h * DH:(h + 1) * DH], ones], axis=1)
                    pv = lax.dot_general(
                        p, v_aug, (((1,), (0,)), ((), ())),
                        preferred_element_type=jnp.float32)
                    blk = pl.ds(h * 128, 128)
                    acc_ref[b, :, blk] = acc_ref[b, :, blk] + pv

        accumulate(my,
                   lambda b: k_ref[b],
                   lambda b: v_ref[b])

        fwd = {}
        for h in range(N_DEV - 1):
            if h == 0:
                rk0.wait()
                rv0.wait()
            else:
                fwd[h].wait()
            if h < N_DEV - 2:
                fwd[h + 1] = make_fwd(h + 1)
                fwd[h + 1].start()
            accumulate((my + N_DEV - h - 1) % N_DEV,
                       lambda b, _h=h: kv_ref[_h, 0, b],
                       lambda b, _h=h: kv_ref[_h, 1, b])

        wo = wo_ref[...].astype(jnp.bfloat16)
        for b in range(B):
            blocks = []
            for h in range(HQ):
                blk = acc_ref[b, :, h * 128:(h + 1) * 128]
                blocks.append(
                    (blk[:, :DH] / blk[:, DH:DH + 1]).astype(jnp.bfloat16))
            ctx = jnp.concatenate(blocks, axis=1)
            out_ref[b] = jnp.dot(ctx, wo, preferred_element_type=jnp.float32)

    return pl.pallas_call(
        body,
        out_shape=jax.ShapeDtypeStruct((B, SQ, DM), jnp.float32),
        in_specs=[pl.BlockSpec(memory_space=pltpu.VMEM)] * 5,
        out_specs=pl.BlockSpec(memory_space=pltpu.VMEM),
        scratch_shapes=[
            pltpu.VMEM((N_DEV - 1, 2, B, SK, HD), jnp.bfloat16),
            pltpu.VMEM((B, SQ, HQ * 128), jnp.float32),
            pltpu.SemaphoreType.DMA((N_DEV - 1,)),
            pltpu.SemaphoreType.DMA((N_DEV - 1,)),
            pltpu.SemaphoreType.DMA((1,)),
            pltpu.SemaphoreType.DMA((1,)),
        ],
        compiler_params=pltpu.CompilerParams(
            collective_id=0, vmem_limit_bytes=60 * 1024 * 1024),
    )(x, Wq, Kb, Vb, Wo)


# device time: 53071 ns/iter; 1.7930x vs baseline; 1.7930x over previous
import jax
import jax.numpy as jnp
from jax import lax
from jax.experimental import pallas as pl
from jax.experimental.pallas import tpu as pltpu

N_DEV = 4
B, SQ, DM = 2, 512, 768
HQ, DH = 8, 64
HD = HQ * DH
SK = 512
PW = HD + 128
NEG = -1e9
SCALE = 0.125
FIXED_MAX = 16.0
NG = 32
NB1 = 128


def kernel(x, Wq, K_ext, V_ext, Wo):
    Kb = K_ext.astype(jnp.bfloat16).reshape(B, SK, HD)
    Vb = V_ext.astype(jnp.bfloat16).reshape(B, SK, HD)

    def body(x_ref, wq_ref, k_ref, v_ref, wo_ref, out_ref,
             p0, p1, p2, p3, ssend, srecv):
        my = lax.axis_index("i")
        pbufs = [p0, p1, p2, p3]

        barrier = pltpu.get_barrier_semaphore()
        for d in range(1, N_DEV):
            pl.semaphore_signal(
                barrier, inc=1,
                device_id=((my + d) % N_DEV,),
                device_id_type=pl.DeviceIdType.MESH,
            )
        pl.semaphore_wait(barrier, N_DEV - 1)

        wq = wq_ref[...].astype(jnp.bfloat16)
        qs = []
        for b in range(B):
            q = jnp.dot(x_ref[b].astype(jnp.bfloat16), wq,
                        preferred_element_type=jnp.float32)
            qs.append((q * SCALE).astype(jnp.bfloat16))

        ones_full = jnp.ones((SK, 64), jnp.bfloat16)

        def pv_block(qrows, krows, vrows, mask):
            s = lax.dot_general(qrows, krows, (((1,), (1,)), ((), ())),
                                preferred_element_type=jnp.float32)
            if mask is not None:
                s = jnp.where(mask, s, NEG)
            p = jnp.exp(s - FIXED_MAX).astype(jnp.bfloat16)
            v_aug = jnp.concatenate(
                [vrows, ones_full[:vrows.shape[0]]], axis=1)
            return lax.dot_general(p, v_aug, (((1,), (0,)), ((), ())),
                                   preferred_element_type=jnp.float32)

        def store(buf, b, r0, h, pv):
            r1 = r0 + pv.shape[0]
            buf[b, r0:r1, h * DH:(h + 1) * DH] = pv[:, :DH].astype(
                jnp.bfloat16)
            buf[b, r0:r1, HD + h:HD + h + 1] = pv[:, DH:DH + 1].astype(
                jnp.bfloat16)

        qi = lax.broadcasted_iota(jnp.int32, (SQ, SK), 0)
        kj = lax.broadcasted_iota(jnp.int32, (SQ, SK), 1)
        mask0 = (jnp.abs(qi - kj) <= 128) | (kj < NG) | (qi < NG)
        bi = lax.broadcasted_iota(jnp.int32, (NB1, NB1), 0)
        bj = lax.broadcasted_iota(jnp.int32, (NB1, NB1), 1)
        mask1 = bj <= bi

        @pl.when(my == 0)
        def _():
            for b in range(B):
                for h in range(HQ):
                    hs = slice(h * DH, (h + 1) * DH)
                    pv = pv_block(qs[b][:, hs], k_ref[b, :, hs],
                                  v_ref[b, :, hs], mask0)
                    store(p0, b, 0, h, pv)

        @pl.when(my == 1)
        def _():
            for b in range(B):
                for h in range(HQ):
                    hs = slice(h * DH, (h + 1) * DH)
                    pv = pv_block(qs[b][:NG, hs], k_ref[b, :, hs],
                                  v_ref[b, :, hs], None)
                    store(p1, b, 0, h, pv)
                    pv = pv_block(qs[b][SQ - NB1:, hs],
                                  k_ref[b, :NB1, hs],
                                  v_ref[b, :NB1, hs], mask1)
                    store(p1, b, NG, h, pv)

        for dev, buf in ((2, p2), (3, p3)):
            @pl.when(my == dev)
            def _(buf=buf):
                for b in range(B):
                    for h in range(HQ):
                        hs = slice(h * DH, (h + 1) * DH)
                        pv = pv_block(qs[b][:NG, hs], k_ref[b, :, hs],
                                      v_ref[b, :, hs], None)
                        store(buf, b, 0, h, pv)

        sends = []
        for s in range(N_DEV):
            @pl.when(my == s)
            def _(s=s):
                for j in range(1, N_DEV):
                    r = pltpu.make_async_remote_copy(
                        src_ref=pbufs[s], dst_ref=pbufs[s],
                        send_sem=ssend.at[j - 1], recv_sem=srecv.at[s],
                        device_id=((my + j) % N_DEV,),
                        device_id_type=pl.DeviceIdType.MESH)
                    r.start()
                    sends.append(r)

        for s in range(N_DEV):
            @pl.when(my != s)
            def _(s=s):
                pltpu.make_async_remote_copy(
                    src_ref=pbufs[s], dst_ref=pbufs[s],
                    send_sem=ssend.at[0], recv_sem=srecv.at[s],
                    device_id=((my + 1) % N_DEV,),
                    device_id_type=pl.DeviceIdType.MESH).wait_recv()

        wo = wo_ref[...].astype(jnp.bfloat16)
        f32 = jnp.float32
        for b in range(B):
            blocks = []
            for h in range(HQ):
                hs = slice(h * DH, (h + 1) * DH)
                ls = slice(HD + h, HD + h + 1)
                ct = (p0[b, :NG, hs].astype(f32) + p1[b, :NG, hs].astype(f32)
                      + p2[b, :, hs].astype(f32) + p3[b, :, hs].astype(f32))
                lt = (p0[b, :NG, ls].astype(f32) + p1[b, :NG, ls].astype(f32)
                      + p2[b, :, ls].astype(f32) + p3[b, :, ls].astype(f32))
                cm = p0[b, NG:SQ - NB1, hs].astype(f32)
                lm = p0[b, NG:SQ - NB1, ls].astype(f32)
                cb = (p0[b, SQ - NB1:, hs].astype(f32)
                      + p1[b, NG:, hs].astype(f32))
                lb = (p0[b, SQ - NB1:, ls].astype(f32)
                      + p1[b, NG:, ls].astype(f32))
                blocks.append(jnp.concatenate(
                    [ct / lt, cm / lm, cb / lb], axis=0).astype(jnp.bfloat16))
            ctx = jnp.concatenate(blocks, axis=1)
            out_ref[b] = jnp.dot(ctx, wo, preferred_element_type=jnp.float32)

        for s in range(N_DEV):
            @pl.when(my == s)
            def _(s=s):
                for j in range(1, N_DEV):
                    pltpu.make_async_remote_copy(
                        src_ref=pbufs[s], dst_ref=pbufs[s],
                        send_sem=ssend.at[j - 1], recv_sem=srecv.at[s],
                        device_id=((my + j) % N_DEV,),
                        device_id_type=pl.DeviceIdType.MESH).wait_send()

    return pl.pallas_call(
        body,
        out_shape=jax.ShapeDtypeStruct((B, SQ, DM), jnp.float32),
        in_specs=[pl.BlockSpec(memory_space=pltpu.VMEM)] * 5,
        out_specs=pl.BlockSpec(memory_space=pltpu.VMEM),
        scratch_shapes=[
            pltpu.VMEM((B, SQ, PW), jnp.bfloat16),
            pltpu.VMEM((B, NG + NB1, PW), jnp.bfloat16),
            pltpu.VMEM((B, NG, PW), jnp.bfloat16),
            pltpu.VMEM((B, NG, PW), jnp.bfloat16),
            pltpu.SemaphoreType.DMA((N_DEV - 1,)),
            pltpu.SemaphoreType.DMA((N_DEV,)),
        ],
        compiler_params=pltpu.CompilerParams(
            collective_id=0, vmem_limit_bytes=60 * 1024 * 1024),
    )(x, Wq, Kb, Vb, Wo)
